# Initial kernel scaffold; baseline (speedup 1.0000x reference)
import jax
import jax.numpy as jnp
from jax.experimental import pallas as pl


def kernel(x, emb_weight):
    raise NotImplementedError("write your pallas kernel here")


# speedup vs baseline: 1.4982x; 1.4982x over previous
import jax
import jax.numpy as jnp
from jax.experimental import pallas as pl


S_BLK = 512


def _add_kernel(x_ref, emb_ref, out_ref):
    out_ref[...] = x_ref[...] + emb_ref[...]


def kernel(x, emb_weight):
    batch, seq_len, emb_dim = x.shape
    grid = (seq_len // S_BLK, batch)
    return pl.pallas_call(
        _add_kernel,
        grid=grid,
        in_specs=[
            pl.BlockSpec((1, S_BLK, emb_dim), lambda i, b: (b, i, 0)),
            pl.BlockSpec((S_BLK, emb_dim), lambda i, b: (i, 0)),
        ],
        out_specs=pl.BlockSpec((1, S_BLK, emb_dim), lambda i, b: (b, i, 0)),
        out_shape=jax.ShapeDtypeStruct(x.shape, x.dtype),
    )(x, emb_weight)


# speedup vs baseline: 1.6681x; 1.1134x over previous
import jax
import jax.numpy as jnp
from jax.experimental import pallas as pl


S_BLK = 1024


def _add_kernel(x_ref, emb_ref, out_ref):
    out_ref[...] = x_ref[...] + emb_ref[...]


def kernel(x, emb_weight):
    batch, seq_len, emb_dim = x.shape
    grid = (seq_len // S_BLK, batch)
    return pl.pallas_call(
        _add_kernel,
        grid=grid,
        in_specs=[
            pl.BlockSpec((1, S_BLK, emb_dim), lambda i, b: (b, i, 0)),
            pl.BlockSpec((S_BLK, emb_dim), lambda i, b: (i, 0)),
        ],
        out_specs=pl.BlockSpec((1, S_BLK, emb_dim), lambda i, b: (b, i, 0)),
        out_shape=jax.ShapeDtypeStruct(x.shape, x.dtype),
    )(x, emb_weight)


# speedup vs baseline: 1.7366x; 1.0410x over previous
import jax
import jax.numpy as jnp
from jax.experimental import pallas as pl


S_BLK = 2048


def _add_kernel(x_ref, emb_ref, out_ref):
    out_ref[...] = x_ref[...] + emb_ref[...]


def kernel(x, emb_weight):
    batch, seq_len, emb_dim = x.shape
    grid = (seq_len // S_BLK, batch)
    return pl.pallas_call(
        _add_kernel,
        grid=grid,
        in_specs=[
            pl.BlockSpec((1, S_BLK, emb_dim), lambda i, b: (b, i, 0)),
            pl.BlockSpec((S_BLK, emb_dim), lambda i, b: (i, 0)),
        ],
        out_specs=pl.BlockSpec((1, S_BLK, emb_dim), lambda i, b: (b, i, 0)),
        out_shape=jax.ShapeDtypeStruct(x.shape, x.dtype),
    )(x, emb_weight)
